# Initial kernel scaffold; baseline (speedup 1.0000x reference)
import jax
import jax.numpy as jnp
from jax.experimental import pallas as pl


def kernel(local_semantic_vectors, input_turns, W, a1, a2):
    raise NotImplementedError("write your pallas kernel here")


# speedup vs baseline: 2.0603x; 2.0603x over previous
import jax
import jax.numpy as jnp
from jax.experimental import pallas as pl
from jax.experimental.pallas import tpu as pltpu

_ALPHA = 0.2
_WINDOW_SIZES = (1, 2, 3)


def _lrelu(x):
    return jnp.where(x >= 0, x, _ALPHA * x)


def _body(turns_ref, x_ref, w_ref, a_ref, o_ref, *, bb, t):
    i = pl.program_id(0)
    y = jnp.dot(x_ref[...], w_ref[...], preferred_element_type=jnp.float32)
    g = jnp.dot(y, a_ref[...], preferred_element_type=jnp.float32)

    iota_n = jax.lax.broadcasted_iota(jnp.int32, (t, t), 0)
    iota_m = jax.lax.broadcasted_iota(jnp.int32, (t, t), 1)
    iota_row = jax.lax.broadcasted_iota(jnp.int32, (1, t), 1)

    for j in range(bb):
        turns = turns_ref[i * bb + j]
        y_j = y[j * t:(j + 1) * t, :]
        g1 = g[j * t:(j + 1) * t, 0:1]
        ga2 = g[j * t:(j + 1) * t, 1:2]

        acc = jnp.zeros((1, y.shape[1]), jnp.float32)
        n_valid_ws = 1
        for ws in _WINDOW_SIZES:
            cnt = jnp.maximum(turns - ws + 2, 0)
            if ws > 1:
                n_valid_ws = n_valid_ws + (cnt > 0).astype(jnp.int32)
            f1 = g1
            f2 = ga2
            for k in range(1, ws):
                f1 = f1 + jnp.concatenate([g1[k:, :], g1[:k, :]], axis=0)
                f2 = f2 + jnp.concatenate([ga2[k:, :], ga2[:k, :]], axis=0)
            f1 = f1 * (1.0 / ws)
            f2 = f2 * (1.0 / ws)
            f2row = f2.T

            validm = iota_row < cnt
            f2max = jnp.max(jnp.where(validm, f2row, -1e30))
            rowmax = _lrelu(f1 + f2max)

            s = jnp.exp(_lrelu(f1 + f2row) - rowmax)
            s = jnp.where((iota_n < cnt) & (iota_m < cnt), s, 0.0)
            r = jnp.sum(s, axis=1, keepdims=True)
            rinv = jnp.where(r > 0, 1.0 / jnp.where(r > 0, r, 1.0), 0.0)
            q = jnp.sum(s * rinv, axis=0, keepdims=True)

            qc = q
            for k in range(1, ws):
                qc = qc + jnp.concatenate([q[:, t - k:], q[:, :t - k]], axis=1)
            cntf = cnt.astype(jnp.float32)
            scale = jnp.where(cnt > 0, 1.0 / jnp.maximum(cntf, 1.0), 0.0)
            wt = qc * (scale * (1.0 / ws))
            acc = acc + jnp.dot(wt, y_j, preferred_element_type=jnp.float32)

        o_ref[j:j + 1, :] = acc / n_valid_ws.astype(jnp.float32)


@jax.jit
def kernel(local_semantic_vectors, input_turns, W, a1, a2):
    T, B, D = local_semantic_vectors.shape
    BB = 8

    x2d = local_semantic_vectors.transpose(1, 0, 2).reshape(B * T, D)
    a12 = jnp.concatenate([a1, a2], axis=1)
    a12 = jnp.pad(a12, ((0, 0), (0, 6)))
    turns = input_turns.astype(jnp.int32)

    import functools
    body = functools.partial(_body, bb=BB, t=T)

    out = pl.pallas_call(
        body,
        grid_spec=pltpu.PrefetchScalarGridSpec(
            num_scalar_prefetch=1,
            grid=(B // BB,),
            in_specs=[
                pl.BlockSpec((BB * T, D), lambda i, s: (i, 0)),
                pl.BlockSpec((D, D), lambda i, s: (0, 0)),
                pl.BlockSpec((D, 8), lambda i, s: (0, 0)),
            ],
            out_specs=pl.BlockSpec((BB, D), lambda i, s: (i, 0)),
        ),
        out_shape=jax.ShapeDtypeStruct((B, D), jnp.float32),
        compiler_params=pltpu.CompilerParams(
            dimension_semantics=("parallel",)),
    )(turns, x2d, W, a12)
    return out


# speedup vs baseline: 2.9737x; 1.4433x over previous
import functools

import jax
import jax.numpy as jnp
from jax.experimental import pallas as pl
from jax.experimental.pallas import tpu as pltpu

_ALPHA = 0.2
_WINDOW_SIZES = (1, 2, 3)


def _body(turns_ref, x_ref, w_ref, a_ref, o_ref, *, bb, t):
    i = pl.program_id(0)
    y = jnp.dot(x_ref[...], w_ref[...], preferred_element_type=jnp.float32)
    g = jnp.dot(y, a_ref[...], preferred_element_type=jnp.float32)
    g2row = g[:, 1:2].T

    iota_row = jax.lax.broadcasted_iota(jnp.int32, (1, t), 1)
    iota_col = jax.lax.broadcasted_iota(jnp.int32, (t, 1), 0)
    ones8 = jnp.ones((t, 8), jnp.float32)

    for j in range(bb):
        turns = turns_ref[i * bb + j]
        y_j = y[j * t:(j + 1) * t, :]
        g1 = g[j * t:(j + 1) * t, 0:1]
        g2 = g2row[0:1, j * t:(j + 1) * t]

        wtot = jnp.zeros((1, t), jnp.float32)
        n_valid_ws = 1
        for ws in _WINDOW_SIZES:
            cnt = jnp.maximum(turns - ws + 2, 0)
            if ws > 1:
                n_valid_ws = n_valid_ws + (cnt > 0).astype(jnp.int32)
            f1 = g1
            f2 = g2
            for k in range(1, ws):
                f1 = f1 + jnp.concatenate([g1[k:, :], g1[:k, :]], axis=0)
                f2 = f2 + jnp.concatenate([g2[:, k:], g2[:, :k]], axis=1)
            f1 = f1 * (1.0 / ws)
            f2 = f2 * (1.0 / ws)

            f2m = jnp.where(iota_row < cnt, f2, -1e30)
            shift0 = jnp.max(f1) + jnp.max(f2m)
            shift = jnp.maximum(shift0, _ALPHA * shift0)

            z = f1 + f2m
            lr = jnp.maximum(z, _ALPHA * z)
            s = jnp.exp(lr - shift)

            r = jnp.dot(s, ones8,
                        preferred_element_type=jnp.float32)[:, 0:1]
            cntf = cnt.astype(jnp.float32)
            sws = jnp.where(cnt > 0,
                            1.0 / (jnp.maximum(cntf, 1.0) * ws), 0.0)
            rinv = jnp.where((iota_col < cnt) & (r > 0), sws / r, 0.0)
            q = jnp.dot(rinv.T, s,
                        preferred_element_type=jnp.float32)

            qc = q
            for k in range(1, ws):
                qc = qc + jnp.concatenate([q[:, t - k:], q[:, :t - k]], axis=1)
            wtot = wtot + qc

        acc = jnp.dot(wtot, y_j, preferred_element_type=jnp.float32)
        o_ref[j:j + 1, :] = acc / n_valid_ws.astype(jnp.float32)


@jax.jit
def kernel(local_semantic_vectors, input_turns, W, a1, a2):
    T, B, D = local_semantic_vectors.shape
    BB = 8

    x2d = local_semantic_vectors.transpose(1, 0, 2).reshape(B * T, D)
    a12 = jnp.concatenate([a1, a2], axis=1)
    a12 = jnp.pad(a12, ((0, 0), (0, 6)))
    turns = input_turns.astype(jnp.int32)

    body = functools.partial(_body, bb=BB, t=T)

    out = pl.pallas_call(
        body,
        grid_spec=pltpu.PrefetchScalarGridSpec(
            num_scalar_prefetch=1,
            grid=(B // BB,),
            in_specs=[
                pl.BlockSpec((BB * T, D), lambda i, s: (i, 0)),
                pl.BlockSpec((D, D), lambda i, s: (0, 0)),
                pl.BlockSpec((D, 8), lambda i, s: (0, 0)),
            ],
            out_specs=pl.BlockSpec((BB, D), lambda i, s: (i, 0)),
        ),
        out_shape=jax.ShapeDtypeStruct((B, D), jnp.float32),
        compiler_params=pltpu.CompilerParams(
            dimension_semantics=("parallel",)),
    )(turns, x2d, W, a12)
    return out


# speedup vs baseline: 5.9099x; 1.9874x over previous
import functools

import jax
import jax.numpy as jnp
from jax.experimental import pallas as pl
from jax.experimental.pallas import tpu as pltpu

_ALPHA = 0.2
_WINDOW_SIZES = (1, 2, 3)


def _tile_rows(a, reps):
    pieces = {1: a}
    p = 1
    while p * 2 <= reps:
        pieces[p * 2] = jnp.concatenate([pieces[p], pieces[p]], axis=0)
        p *= 2
    out = []
    rem = reps
    for size in sorted(pieces, reverse=True):
        while rem >= size:
            out.append(pieces[size])
            rem -= size
    return jnp.concatenate(out, axis=0) if len(out) > 1 else out[0]


def _body(x_ref, t_ref, w_ref, a_ref, o_ref, *, bb, t):
    n = t * bb
    x2 = x_ref[...].reshape(n, x_ref.shape[-1])
    y = jnp.dot(x2, w_ref[...], preferred_element_type=jnp.float32)
    g = jnp.dot(y, a_ref[...], preferred_element_type=jnp.float32)
    g1c = g[:, 0:1]
    g2s = jnp.reshape(g[:, 1:2], (t, bb)).T

    turns = t_ref[...]
    lane = jax.lax.broadcasted_iota(jnp.int32, (bb, t), 1)
    row_i = jax.lax.broadcasted_iota(jnp.int32, (n, 1), 0)
    trow = row_i // bb
    l2 = jax.lax.broadcasted_iota(jnp.int32, (bb, n), 1)
    s2 = jax.lax.broadcasted_iota(jnp.int32, (bb, n), 0)
    P = ((l2 % bb) == s2).astype(jnp.float32)
    l3 = jax.lax.broadcasted_iota(jnp.int32, (n, bb), 1)
    s3 = jax.lax.broadcasted_iota(jnp.int32, (n, bb), 0)
    PT = ((s3 % bb) == l3).astype(jnp.float32)
    lane2 = jax.lax.broadcasted_iota(jnp.int32, (n, t), 1)
    tind = (lane2 == trow).astype(jnp.float32)
    ones8 = jnp.ones((t, 8), jnp.float32)

    omega = jnp.zeros((bb, t), jnp.float32)
    nws = jnp.ones((bb, 1), jnp.float32)
    for ws in _WINDOW_SIZES:
        cnt = jnp.maximum(turns - (ws - 2), 0)
        if ws > 1:
            nws = nws + (cnt > 0).astype(jnp.float32)
        f1 = g1c
        f2 = g2s
        for k in range(1, ws):
            kb = k * bb
            f1 = f1 + jnp.concatenate([g1c[kb:, :], g1c[:kb, :]], axis=0)
            f2 = f2 + jnp.concatenate([g2s[:, k:], g2s[:, :k]], axis=1)
        inv_ws = 1.0 / ws
        f1 = f1 * inv_ws
        f2 = f2 * inv_ws

        f2m = jnp.where(lane < cnt, f2, -1e30)
        m2 = jnp.max(f2m, axis=1, keepdims=True)

        F2 = _tile_rows(f2m, t)
        m2r = _tile_rows(m2, t)
        cntr = _tile_rows(cnt, t)

        zmax = f1 + m2r
        shift = jnp.maximum(zmax, _ALPHA * zmax)
        z = f1 + F2
        lr = jnp.maximum(z, _ALPHA * z)
        s = jnp.exp(lr - shift)

        r = jnp.dot(s, ones8,
                    preferred_element_type=jnp.float32)[:, 0:1]
        denom = r * cntr.astype(jnp.float32) * float(ws)
        rinv = jnp.where(trow < cntr, 1.0 / denom, 0.0)
        q = jnp.dot(P, s * rinv,
                    preferred_element_type=jnp.float32)

        qc = q
        for k in range(1, ws):
            qc = qc + jnp.concatenate([q[:, t - k:], q[:, :t - k]], axis=1)
        omega = omega + qc

    omega = omega / nws
    wexp = jnp.dot(PT, omega, preferred_element_type=jnp.float32)
    wcol = jnp.dot(wexp * tind, ones8,
                   preferred_element_type=jnp.float32)[:, 0:1]
    out = jnp.dot(P, wcol * y, preferred_element_type=jnp.float32)
    o_ref[...] = out


@jax.jit
def kernel(local_semantic_vectors, input_turns, W, a1, a2):
    T, B, D = local_semantic_vectors.shape
    BB = 8

    a12 = jnp.concatenate([a1, a2], axis=1)
    a12 = jnp.pad(a12, ((0, 0), (0, 6)))
    turns2 = input_turns.astype(jnp.int32).reshape(B, 1)

    body = functools.partial(_body, bb=BB, t=T)

    out = pl.pallas_call(
        body,
        grid=(B // BB,),
        in_specs=[
            pl.BlockSpec((T, BB, D), lambda i: (0, i, 0)),
            pl.BlockSpec((BB, 1), lambda i: (i, 0)),
            pl.BlockSpec((D, D), lambda i: (0, 0)),
            pl.BlockSpec((D, 8), lambda i: (0, 0)),
        ],
        out_specs=pl.BlockSpec((BB, D), lambda i: (i, 0)),
        out_shape=jax.ShapeDtypeStruct((B, D), jnp.float32),
        compiler_params=pltpu.CompilerParams(
            dimension_semantics=("parallel",)),
    )(local_semantic_vectors, turns2, W, a12)
    return out


# speedup vs baseline: 6.2230x; 1.0530x over previous
import functools

import jax
import jax.numpy as jnp
from jax.experimental import pallas as pl
from jax.experimental.pallas import tpu as pltpu

_ALPHA = 0.2
_WINDOW_SIZES = (1, 2, 3)


def _body(x_ref, t_ref, w_ref, a_ref, o_ref, *, bb, t):
    n = t * bb
    tp = t + 1
    nw = len(_WINDOW_SIZES)
    x2 = x_ref[...].reshape(n, x_ref.shape[-1])
    y = jnp.dot(x2, w_ref[...], preferred_element_type=jnp.float32)
    g = jnp.dot(y, a_ref[...], preferred_element_type=jnp.float32)
    g1c = g[:, 0:1]
    g2s = jnp.reshape(g[:, 1:2], (t, bb)).T

    turns = t_ref[...]
    lane = jax.lax.broadcasted_iota(jnp.int32, (bb, t), 1)
    l2 = jax.lax.broadcasted_iota(jnp.int32, (bb, n), 1)
    s2 = jax.lax.broadcasted_iota(jnp.int32, (bb, n), 0)
    Pm = (l2 % bb) == s2
    trowl = l2 // bb
    l3 = jax.lax.broadcasted_iota(jnp.int32, (n, bb), 1)
    s3 = jax.lax.broadcasted_iota(jnp.int32, (n, bb), 0)
    PT = ((s3 % bb) == l3).astype(jnp.float32)

    rolls = [g1c]
    for k in range(1, nw):
        kb = k * bb
        rolls.append(jnp.concatenate([g1c[kb:, :], g1c[:kb, :]], axis=0))
    L = jnp.concatenate([PT] + rolls, axis=1)

    rs_l = jax.lax.broadcasted_iota(jnp.int32, (tp, 8), 0)
    ones_rs = (rs_l < t).astype(jnp.float32)

    omega = jnp.zeros((bb, t), jnp.float32)
    nws = jnp.ones((bb, 1), jnp.float32)
    for ws in _WINDOW_SIZES:
        cnt = jnp.maximum(turns - (ws - 2), 0)
        if ws > 1:
            nws = nws + (cnt > 0).astype(jnp.float32)
        f2 = g2s
        for k in range(1, ws):
            f2 = f2 + jnp.concatenate([g2s[:, k:], g2s[:, :k]], axis=1)
        f2 = f2 * (1.0 / ws)
        f2m = jnp.where(lane < cnt, f2, -1e30)
        m2 = jnp.max(f2m, axis=1, keepdims=True)

        top = jnp.concatenate([f2m, m2], axis=1)
        wr_s = jax.lax.broadcasted_iota(jnp.int32, (nw, tp), 0)
        wrows = jnp.where(wr_s < ws, 1.0 / ws, 0.0)
        R = jnp.concatenate([top, wrows], axis=0)

        zf = jnp.dot(L, R, preferred_element_type=jnp.float32)
        zmax = zf[:, t:tp]
        shift = jnp.maximum(zmax, _ALPHA * zmax)
        lr = jnp.maximum(zf, _ALPHA * zf)
        s = jnp.exp(lr - shift)

        r8 = jnp.dot(s, ones_rs,
                     preferred_element_type=jnp.float32)
        rT = jnp.transpose(r8)
        denom = rT * (cnt.astype(jnp.float32) * float(ws))
        Pr = jnp.where(Pm & (trowl < cnt), 1.0 / denom, 0.0)
        q = jnp.dot(Pr, s, preferred_element_type=jnp.float32)

        qv = q[:, :t]
        qc = qv
        for k in range(1, ws):
            qc = qc + jnp.concatenate([qv[:, t - k:], qv[:, :t - k]], axis=1)
        omega = omega + qc

    omega = omega / nws
    omega_exp = jnp.repeat(omega, bb, axis=1)
    Wm = jnp.where(Pm, omega_exp, 0.0)
    out = jnp.dot(Wm, y, preferred_element_type=jnp.float32)
    o_ref[...] = out


@jax.jit
def kernel(local_semantic_vectors, input_turns, W, a1, a2):
    T, B, D = local_semantic_vectors.shape
    BB = 8

    a12 = jnp.concatenate([a1, a2], axis=1)
    a12 = jnp.pad(a12, ((0, 0), (0, 6)))
    turns2 = input_turns.astype(jnp.int32).reshape(B, 1)

    body = functools.partial(_body, bb=BB, t=T)

    out = pl.pallas_call(
        body,
        grid=(B // BB,),
        in_specs=[
            pl.BlockSpec((T, BB, D), lambda i: (0, i, 0)),
            pl.BlockSpec((BB, 1), lambda i: (i, 0)),
            pl.BlockSpec((D, D), lambda i: (0, 0)),
            pl.BlockSpec((D, 8), lambda i: (0, 0)),
        ],
        out_specs=pl.BlockSpec((BB, D), lambda i: (i, 0)),
        out_shape=jax.ShapeDtypeStruct((B, D), jnp.float32),
        compiler_params=pltpu.CompilerParams(
            dimension_semantics=("parallel",)),
    )(local_semantic_vectors, turns2, W, a12)
    return out


# speedup vs baseline: 7.9141x; 1.2717x over previous
import functools

import jax
import jax.numpy as jnp
from jax.experimental import pallas as pl
from jax.experimental.pallas import tpu as pltpu

_ALPHA = 0.2
_WINDOW_SIZES = (1, 2, 3)


def _body(x_ref, t_ref, w_ref, a_ref, o_ref, *, bb, t):
    n = t * bb
    tp = t + 1
    nw = len(_WINDOW_SIZES)
    x2 = x_ref[...].reshape(n, x_ref.shape[-1])
    y = jnp.dot(x2, w_ref[...], preferred_element_type=jnp.float32)
    g = jnp.dot(x2, a_ref[...], preferred_element_type=jnp.float32)
    gT = jnp.transpose(g)

    turns = t_ref[...]
    lane = jax.lax.broadcasted_iota(jnp.int32, (bb, t), 1)
    l2 = jax.lax.broadcasted_iota(jnp.int32, (bb, n), 1)
    s2 = jax.lax.broadcasted_iota(jnp.int32, (bb, n), 0)
    Pm = (l2 % bb) == s2
    trowl = l2 // bb
    l3 = jax.lax.broadcasted_iota(jnp.int32, (n, bb), 1)
    s3 = jax.lax.broadcasted_iota(jnp.int32, (n, bb), 0)
    PT = ((s3 % bb) == l3).astype(jnp.float32)
    l4 = jax.lax.broadcasted_iota(jnp.int32, (n, t), 1)
    s4 = jax.lax.broadcasted_iota(jnp.int32, (n, t), 0)
    tind = (l4 == s4 // bb).astype(jnp.float32)

    A2 = jnp.where(Pm, jnp.broadcast_to(gT[1:2, :], (bb, n)), 0.0)
    g2s = jnp.dot(A2, tind, preferred_element_type=jnp.float32)

    g1r = gT[0:1, :]
    g1rows = [g1r]
    for k in range(1, nw):
        kb = k * bb
        g1rows.append(jnp.concatenate([g1r[:, kb:], g1r[:, :kb]], axis=1))
    g1cols = jnp.transpose(jnp.concatenate(g1rows, axis=0))
    L = jnp.concatenate([PT, g1cols], axis=1)

    rs_l = jax.lax.broadcasted_iota(jnp.int32, (tp, bb), 0)
    ones_rs = (rs_l < t).astype(jnp.float32)

    omega = jnp.zeros((bb, t), jnp.float32)
    nws = jnp.ones((bb, 1), jnp.float32)
    for ws in _WINDOW_SIZES:
        cnt = jnp.maximum(turns - (ws - 2), 0)
        if ws > 1:
            nws = nws + (cnt > 0).astype(jnp.float32)
        f2 = g2s
        for k in range(1, ws):
            f2 = f2 + jnp.concatenate([g2s[:, k:], g2s[:, :k]], axis=1)
        f2 = f2 * (1.0 / ws)
        f2m = jnp.where(lane < cnt, f2, -1e30)
        m2 = jnp.max(f2m, axis=1, keepdims=True)

        top = jnp.concatenate([f2m, m2], axis=1)
        wr_s = jax.lax.broadcasted_iota(jnp.int32, (nw, tp), 0)
        wrows = jnp.where(wr_s < ws, 1.0 / ws, 0.0)
        R = jnp.concatenate([top, wrows], axis=0)

        zf = jnp.dot(L, R, preferred_element_type=jnp.float32)
        zmax = zf[:, t:tp]
        shift = jnp.maximum(zmax, _ALPHA * zmax)
        lr = jnp.maximum(zf, _ALPHA * zf)
        s = jnp.exp(lr - shift)

        r8 = jnp.dot(s, ones_rs,
                     preferred_element_type=jnp.float32)
        rT = jnp.transpose(r8)
        denom = rT * (cnt.astype(jnp.float32) * float(ws))
        Pr = jnp.where(Pm & (trowl < cnt), 1.0 / denom, 0.0)
        q = jnp.dot(Pr, s, preferred_element_type=jnp.float32)

        qv = q[:, :t]
        qc = qv
        for k in range(1, ws):
            qc = qc + jnp.concatenate([qv[:, t - k:], qv[:, :t - k]], axis=1)
        omega = omega + qc

    omega = omega / nws
    wrow = jnp.dot(tind, jnp.transpose(omega),
                   preferred_element_type=jnp.float32)
    Wm = jnp.transpose(wrow * PT)
    out = jnp.dot(Wm, y, preferred_element_type=jnp.float32)
    o_ref[...] = out


@jax.jit
def kernel(local_semantic_vectors, input_turns, W, a1, a2):
    T, B, D = local_semantic_vectors.shape
    BB = 32

    a12 = jnp.concatenate([a1, a2], axis=1)
    wa = jnp.pad(jnp.dot(W, a12), ((0, 0), (0, 6)))
    turns2 = input_turns.astype(jnp.int32).reshape(B, 1)

    body = functools.partial(_body, bb=BB, t=T)

    out = pl.pallas_call(
        body,
        grid=(B // BB,),
        in_specs=[
            pl.BlockSpec((T, BB, D), lambda i: (0, i, 0)),
            pl.BlockSpec((BB, 1), lambda i: (i, 0)),
            pl.BlockSpec((D, D), lambda i: (0, 0)),
            pl.BlockSpec((D, 8), lambda i: (0, 0)),
        ],
        out_specs=pl.BlockSpec((BB, D), lambda i: (i, 0)),
        out_shape=jax.ShapeDtypeStruct((B, D), jnp.float32),
        compiler_params=pltpu.CompilerParams(
            dimension_semantics=("parallel",)),
    )(local_semantic_vectors, turns2, W, wa)
    return out
